# baseline (speedup 1.0000x reference)
import functools

import jax
import jax.numpy as jnp
from jax import lax
from jax.experimental import pallas as pl
from jax.experimental.pallas import tpu as pltpu
from jax.experimental.pallas import tpu_sc as plsc

EMB_DIM = 64
NC = 2
NS = 16
NW = NC * NS
CHUNK = 512


@functools.partial(jax.jit, static_argnames=("total",))
def _embed(table, flat_idx, *, total):
    b_per_w = total // NW
    n_chunks = b_per_w // CHUNK

    def body(table_hbm, idx_hbm, out_hbm, idx_v, rows_v, sem):
        wid = lax.axis_index("s") * NC + lax.axis_index("c")
        base = wid * b_per_w

        def step(g, _):
            off = base + g * CHUNK
            pltpu.sync_copy(idx_hbm.at[pl.ds(off, CHUNK)], idx_v)
            pltpu.async_copy(table_hbm.at[idx_v], rows_v, sem).wait()
            pltpu.sync_copy(rows_v, out_hbm.at[pl.ds(off, CHUNK)])
            return 0

        lax.fori_loop(0, n_chunks, step, 0)

    run = pl.kernel(
        body,
        out_type=jax.ShapeDtypeStruct((total, EMB_DIM), jnp.float32),
        mesh=plsc.VectorSubcoreMesh(
            core_axis_name="c", subcore_axis_name="s",
            num_cores=NC, num_subcores=NS,
        ),
        scratch_types=[
            pltpu.VMEM((CHUNK,), jnp.int32),
            pltpu.VMEM((CHUNK, EMB_DIM), jnp.float32),
            pltpu.SemaphoreType.DMA,
        ],
        compiler_params=pltpu.CompilerParams(use_tc_tiling_on_sc=False),
    )
    return run(table, flat_idx)


def kernel(x, emb_weight):
    b, l = x.shape
    total = b * l
    flat = x.reshape(total).astype(jnp.int32)
    out = _embed(emb_weight, flat, total=total)
    return out.reshape(b, l, EMB_DIM)


# speedup vs baseline: 1.0439x; 1.0439x over previous
import functools

import jax
import jax.numpy as jnp
from jax import lax
from jax.experimental import pallas as pl
from jax.experimental.pallas import tpu as pltpu
from jax.experimental.pallas import tpu_sc as plsc

EMB_DIM = 64
NC = 2
NS = 16
NW = NC * NS
CHUNK = 400
NBUF = 4
LOOKAHEAD = 2


@functools.partial(jax.jit, static_argnames=("total",))
def _embed(table, flat_idx, *, total):
    b_per_w = total // NW
    n_chunks = b_per_w // CHUNK
    n_groups = n_chunks // NBUF

    def body(table_hbm, idx_hbm, out_hbm, idx_all,
             r0, r1, r2, r3, g0, g1, g2, g3, o0, o1, o2, o3):
        rows = (r0, r1, r2, r3)
        gsem = (g0, g1, g2, g3)
        osem = (o0, o1, o2, o3)
        wid = lax.axis_index("s") * NC + lax.axis_index("c")
        base = wid * b_per_w

        pltpu.sync_copy(idx_hbm.at[pl.ds(base, b_per_w)], idx_all)

        def start_gather(c, j):
            idx = idx_all.at[pl.ds(c * CHUNK, CHUNK)]
            pltpu.async_copy(table_hbm.at[idx], rows[j], gsem[j])

        def start_out(c, j):
            dst = out_hbm.at[pl.ds(base + c * CHUNK, CHUNK)]
            pltpu.async_copy(rows[j], dst, osem[j])

        def wait_gather(j):
            idx = idx_all.at[pl.ds(0, CHUNK)]
            pltpu.make_async_copy(table_hbm.at[idx], rows[j], gsem[j]).wait()

        def wait_out(c, j):
            dst = out_hbm.at[pl.ds(base + c * CHUNK, CHUNK)]
            pltpu.make_async_copy(rows[j], dst, osem[j]).wait()

        def slot(c, j, first, last):
            cn = c + LOOKAHEAD
            jn = (j + LOOKAHEAD) % NBUF
            if not last:
                if not first:
                    wait_out(cn - NBUF, jn)
                start_gather(cn, jn)
            wait_gather(j)
            start_out(c, j)

        for j in range(LOOKAHEAD):
            start_gather(j, j)

        for j in range(NBUF):
            slot(j, j, first=(j < NBUF - LOOKAHEAD), last=False)

        def group(t, _):
            for j in range(NBUF):
                slot(t * NBUF + j, j, first=False, last=False)
            return 0

        lax.fori_loop(1, n_groups - 1, group, 0)

        for j in range(NBUF):
            c = (n_groups - 1) * NBUF + j
            slot(c, j, first=False, last=(c + LOOKAHEAD >= n_chunks))
        for j in range(NBUF):
            wait_out((n_groups - 1) * NBUF + j, j)

    run = pl.kernel(
        body,
        out_type=jax.ShapeDtypeStruct((total, EMB_DIM), jnp.float32),
        mesh=plsc.VectorSubcoreMesh(
            core_axis_name="c", subcore_axis_name="s",
            num_cores=NC, num_subcores=NS,
        ),
        scratch_types=(
            [pltpu.VMEM((b_per_w,), jnp.int32)]
            + [pltpu.VMEM((CHUNK, EMB_DIM), jnp.float32)] * NBUF
            + [pltpu.SemaphoreType.DMA] * (2 * NBUF)
        ),
        compiler_params=pltpu.CompilerParams(use_tc_tiling_on_sc=False),
    )
    return run(table, flat_idx)


def kernel(x, emb_weight):
    b, l = x.shape
    total = b * l
    flat = x.reshape(total).astype(jnp.int32)
    out = _embed(emb_weight, flat, total=total)
    return out.reshape(b, l, EMB_DIM)
